# baseline (speedup 1.0000x reference)
import functools

import jax
import jax.numpy as jnp
import numpy as np
from jax import lax
from jax.experimental import pallas as pl
from jax.experimental.pallas import tpu as pltpu
from jax.experimental.pallas import tpu_sc as plsc

_B = 32
_PACK = 4
_L = _B * _PACK
_D = _B - 1
_LD = _D * _PACK
_LOWER = -3.0
_MIN_BIN = 0.01
_MIN_DERIV = 0.01
_RED_RANGE = 6.0 - _B * _MIN_BIN
_BLOCK = 2048

_UPPER_CONST = float(
    np.float32(np.float32(_RED_RANGE) * np.float32(1.0)
               + np.float32(np.float32(_MIN_BIN) * np.float32(_B)))
    + np.float32(_LOWER))

_HI = lax.Precision.HIGHEST
_LO = lax.Precision.DEFAULT


def _seg_matrices():
    i = np.arange(_L)
    seg = i[:, None] // _B == i[None, :] // _B
    bd_lt = (seg & (i[:, None] % _B <= i[None, :] % _B)).astype(np.float32)
    bd_ones = seg.astype(np.float32)
    lts = np.concatenate([bd_lt, bd_ones], axis=1)
    e4 = (np.arange(_PACK)[:, None] == i[None, :] // _B).astype(np.float32)
    j = np.arange(_LD)
    m2 = (i[:, None] // _B == j[None, :] // _D).astype(np.float32)
    cn2 = np.concatenate([bd_ones, m2], axis=1)
    m31 = (j[:, None] // _D == i[None, :] // _B).astype(np.float32)
    c1 = (i[:, None] == _B * np.arange(_PACK)[None, :]).astype(np.float32)
    return (jnp.asarray(lts), jnp.asarray(bd_ones), jnp.asarray(e4),
            jnp.asarray(cn2), jnp.asarray(m31), jnp.asarray(c1))


def _mm(a, b, precision=_HI):
    return lax.dot_general(a, b, (((1,), (0,)), ((), ())),
                           preferred_element_type=jnp.float32,
                           precision=precision)


def _spline_body(x_ref, ubx_ref, uby_ref, ud_ref,
                 lts_ref, on_ref, e4_ref, cn2_ref, m31_ref, c1_ref,
                 vals_ref, acc_ref):
    i = pl.program_id(0)
    x4 = x_ref[:, :]
    ubx = ubx_ref[:, :]
    uby = uby_ref[:, :]
    ud = ud_ref[:, :]
    bd_ones = on_ref[:, :]
    rows = ubx.shape[0]

    il = lax.broadcasted_iota(jnp.int32, (rows, _L), 1)
    kf = jnp.bitwise_and(il, _B - 1).astype(jnp.float32)
    kterm = _MIN_BIN * (kf + 1.0) + _LOWER

    xb = _mm(x4, e4_ref[:, :])

    ex = jnp.exp(ubx)
    ey = jnp.exp(uby)
    cs2x = _mm(ex, lts_ref[:, :])
    cs2y = _mm(ey, lts_ref[:, :])
    totx = cs2x[:, _L:]
    toty = cs2y[:, _L:]
    posx = _RED_RANGE * (cs2x[:, :_L] / totx) + kterm
    cqy = _RED_RANGE * (cs2y[:, :_L] / toty) + kterm

    in_range = jnp.logical_and(xb > _LOWER, xb < _UPPER_CONST)
    xs = jnp.clip(xb, _LOWER, _UPPER_CONST)

    c = (posx < xs).astype(jnp.float32)
    cn2 = _mm(c, cn2_ref[:, :], _LO)
    cnt = cn2[:, :_L]
    cnt124 = cn2[:, _L:]

    oh_lo = (kf == cnt - 1.0).astype(jnp.float32)
    oh_hi = (kf == cnt).astype(jnp.float32)
    first = cnt == 0.0
    last = cnt == jnp.float32(_B - 1)

    lox = _mm(posx * oh_lo, bd_ones)
    ylo = _mm(cqy * oh_lo, bd_ones)
    lower_x = jnp.where(first, jnp.float32(_LOWER), lox)
    lower_y = jnp.where(first, jnp.float32(_LOWER), ylo)

    dex = _mm(ex * oh_hi, bd_ones, _LO)
    dey = _mm(ey * oh_hi, bd_ones, _LO)
    delta_x = _RED_RANGE * (dex / totx) + _MIN_BIN
    delta_y = jnp.where(last, jnp.float32(_UPPER_CONST) - lower_y,
                        _RED_RANGE * (dey / toty) + _MIN_BIN)

    il124 = lax.broadcasted_iota(jnp.int32, (rows, _LD), 1)
    j31 = (il124 - _D * (il124 // _D)).astype(jnp.float32)
    ohd_lo = (j31 == cnt124 - 1.0).astype(jnp.float32)
    ohd_hi = (j31 == cnt124).astype(jnp.float32)
    ulo = _mm(ud * ohd_lo, m31_ref[:, :], _LO)
    uhi = _mm(ud * ohd_hi, m31_ref[:, :], _LO)

    def softplus(v):
        return jnp.maximum(v, 0.0) + jnp.log(1.0 + jnp.exp(-jnp.abs(v)))

    lower_d = jnp.where(first, 1.0, softplus(ulo) + _MIN_DERIV)
    upper_d = jnp.where(last, 1.0, softplus(uhi) + _MIN_DERIV)

    r_dx = 1.0 / delta_x
    slope = delta_y * r_dx
    alpha = jnp.clip((xs - lower_x) * r_dx, 0.0, 1.0)
    a2 = alpha * alpha
    om = 1.0 - alpha
    beta = alpha * om
    gamma = om * om
    eps = upper_d + lower_d - 2.0 * slope
    r_den = 1.0 / (slope + eps * beta)
    val_s = lower_y + delta_y * (slope * a2 + lower_d * beta) * r_den
    der_s = slope * slope * (upper_d * a2 + 2.0 * slope * beta
                             + lower_d * gamma) * (r_den * r_den)

    val = jnp.where(in_range, val_s, xb)
    der = jnp.where(in_range, der_s, 1.0)

    vals_ref[:, :] = _mm(val * (kf == 0.0).astype(jnp.float32), c1_ref[:, :])

    part = jnp.sum(jnp.log(jnp.abs(der)), axis=0, keepdims=True)
    part = jnp.sum(part, axis=1, keepdims=True) * (1.0 / _B)

    @pl.when(i == 0)
    def _init():
        acc_ref[:, :] = jnp.zeros((1, 1), jnp.float32)

    acc_ref[:, :] += part


_NSC = 8192
_NW = 32
_SCL = 16
_ROWS_W = _NSC // _NW
_CHUNKS_W = _ROWS_W // _SCL

_LN2 = float(np.log(2.0))
_SQRT2 = float(np.sqrt(2.0))


def _sc_log(y):
    bits = plsc.bitcast(y, jnp.int32)
    e = jnp.right_shift(bits, 23) - 127
    m = plsc.bitcast(jnp.bitwise_or(jnp.bitwise_and(bits, 0x007FFFFF),
                                    0x3F800000), jnp.float32)
    big = m > _SQRT2
    m = jnp.where(big, m * 0.5, m)
    ef = e.astype(jnp.float32) + jnp.where(big, 1.0, 0.0)
    t = (m - 1.0) / (m + 1.0)
    t2 = t * t
    p = 1.0 + t2 * (1.0 / 3.0 + t2 * (1.0 / 5.0 + t2 * (1.0 / 7.0
                                                        + t2 / 9.0)))
    return 2.0 * t * p + _LN2 * ef


def _sc_softplus(v):
    return jnp.maximum(v, 0.0) + _sc_log(1.0 + jnp.exp(-jnp.abs(v)))


def _sc_spline(x_hbm, ubx_hbm, uby_hbm, ud_hbm, vals_hbm, part_hbm,
               xv, ubv, ubyv, udv, valv, partv):
    wid = lax.axis_index("s") * 2 + lax.axis_index("c")
    iota = lax.broadcasted_iota(jnp.int32, (_SCL,), 0)
    row_off32 = iota * _B
    row_off31 = iota * _D
    zeros = jnp.zeros((_SCL,), jnp.float32)

    def chunk_body(cki, acc):
        row0 = wid * _ROWS_W + cki * _SCL
        pltpu.sync_copy(x_hbm.at[pl.ds(row0, _SCL)], xv)
        pltpu.sync_copy(ubx_hbm.at[pl.ds(row0, _SCL)], ubv)
        pltpu.sync_copy(uby_hbm.at[pl.ds(row0, _SCL)], ubyv)
        pltpu.sync_copy(ud_hbm.at[pl.ds(row0, _SCL)], udv)

        xvec = xv[...]

        def tot_body(k, carry):
            tx, ty = carry
            ksp = iota * 0 + k
            exk = jnp.exp(plsc.load_gather(ubv, [iota, ksp]))
            eyk = jnp.exp(plsc.load_gather(ubyv, [iota, ksp]))
            return tx + exk, ty + eyk

        totx, toty = lax.fori_loop(0, _B, tot_body, (zeros, zeros))
        rtx = _RED_RANGE / totx
        rty = _RED_RANGE / toty

        xs = jnp.clip(xvec, _LOWER, _UPPER_CONST)
        in_range = jnp.logical_and(xvec > _LOWER, xvec < _UPPER_CONST)

        def scan_body(k, carry):
            (cumx, cumy, cnt, low_x, low_y, dex, dey, found) = carry
            ksp = iota * 0 + k
            exk = jnp.exp(plsc.load_gather(ubv, [iota, ksp]))
            eyk = jnp.exp(plsc.load_gather(ubyv, [iota, ksp]))
            cumx = cumx + exk
            cumy = cumy + eyk
            kshift = _MIN_BIN * (k + 1).astype(jnp.float32) + _LOWER
            pos_k = cumx * rtx + kshift
            cq_k = cumy * rty + kshift
            cmask = pos_k < xs
            cnt = cnt + jnp.where(cmask, 1.0, 0.0)
            low_x = jnp.where(cmask, pos_k, low_x)
            low_y = jnp.where(cmask, cq_k, low_y)
            newly = jnp.logical_and(jnp.logical_not(cmask), found == 0.0)
            dex = jnp.where(newly, exk * rtx + _MIN_BIN, dex)
            dey = jnp.where(newly, eyk * rty + _MIN_BIN, dey)
            found = jnp.where(newly, 1.0, found)
            return (cumx, cumy, cnt, low_x, low_y, dex, dey, found)

        init = (zeros, zeros, zeros, zeros + _LOWER, zeros + _LOWER,
                zeros + 1.0, zeros + 1.0, zeros)
        (_, _, cnt, lower_x, lower_y, delta_x, dey, _) = lax.fori_loop(
            0, _B, scan_body, init)

        cnti = cnt.astype(jnp.int32)
        first = cnt == 0.0
        last = cnt == jnp.float32(_B - 1)
        delta_y = jnp.where(last, _UPPER_CONST - lower_y, dey)

        idx_lo = jnp.maximum(cnti - 1, 0)
        idx_hi = jnp.minimum(cnti, _D - 1)
        ulo = plsc.load_gather(udv, [iota, idx_lo])
        uhi = plsc.load_gather(udv, [iota, idx_hi])
        lower_d = jnp.where(first, 1.0, _sc_softplus(ulo) + _MIN_DERIV)
        upper_d = jnp.where(last, 1.0, _sc_softplus(uhi) + _MIN_DERIV)

        r_dx = 1.0 / delta_x
        slope = delta_y * r_dx
        alpha = jnp.clip((xs - lower_x) * r_dx, 0.0, 1.0)
        a2 = alpha * alpha
        om = 1.0 - alpha
        beta = alpha * om
        gamma = om * om
        eps = upper_d + lower_d - 2.0 * slope
        r_den = 1.0 / (slope + eps * beta)
        val_s = lower_y + delta_y * (slope * a2 + lower_d * beta) * r_den
        der_s = slope * slope * (upper_d * a2 + 2.0 * slope * beta
                                 + lower_d * gamma) * (r_den * r_den)
        val = jnp.where(in_range, val_s, xvec)
        der = jnp.where(in_range, der_s, 1.0)

        valv[...] = val
        pltpu.sync_copy(valv, vals_hbm.at[pl.ds(row0, _SCL)])
        return acc + _sc_log(jnp.abs(der))

    acc = lax.fori_loop(0, _CHUNKS_W, chunk_body, zeros)
    partv[...] = acc
    pltpu.sync_copy(partv, part_hbm.at[wid])


def _sc_call(x_sc, ubx_flat, uby_flat, ud_flat):
    mesh = plsc.VectorSubcoreMesh(core_axis_name="c", subcore_axis_name="s")
    f = functools.partial(
        pl.kernel, mesh=mesh,
        compiler_params=pltpu.CompilerParams(needs_layout_passes=False),
        out_type=[jax.ShapeDtypeStruct((_NSC,), jnp.float32),
                  jax.ShapeDtypeStruct((_NW, _SCL), jnp.float32)],
        scratch_types=[
            pltpu.VMEM((_SCL,), jnp.float32),
            pltpu.VMEM((_SCL, _B), jnp.float32),
            pltpu.VMEM((_SCL, _B), jnp.float32),
            pltpu.VMEM((_SCL, _D), jnp.float32),
            pltpu.VMEM((_SCL,), jnp.float32),
            pltpu.VMEM((_SCL,), jnp.float32),
        ],
    )(_sc_spline)
    return f(x_sc, ubx_flat, uby_flat, ud_flat)


@jax.jit
def kernel(x, unconst_bin_size_x, unconst_bin_size_y, unconst_derivs):
    n = x.shape[0]

    vals_sc, parts_sc = _sc_call(
        x[:_NSC],
        unconst_bin_size_x[:_NSC],
        unconst_bin_size_y[:_NSC],
        unconst_derivs[:_NSC])

    ntc = n - _NSC
    np4 = ntc // _PACK
    r = _BLOCK
    grid = np4 // r
    x4 = x[_NSC:].reshape(np4, _PACK)
    ubx = unconst_bin_size_x[_NSC:].reshape(np4, _L)
    uby = unconst_bin_size_y[_NSC:].reshape(np4, _L)
    ud = unconst_derivs[_NSC:].reshape(np4, _LD)
    mats = _seg_matrices()

    const_spec = [
        pl.BlockSpec(m.shape, lambda i: (0, 0)) for m in mats
    ]
    vals, acc = pl.pallas_call(
        _spline_body,
        grid=(grid,),
        in_specs=[
            pl.BlockSpec((r, _PACK), lambda i: (i, 0)),
            pl.BlockSpec((r, _L), lambda i: (i, 0)),
            pl.BlockSpec((r, _L), lambda i: (i, 0)),
            pl.BlockSpec((r, _LD), lambda i: (i, 0)),
        ] + const_spec,
        out_specs=[
            pl.BlockSpec((r, _PACK), lambda i: (i, 0)),
            pl.BlockSpec((1, 1), lambda i: (0, 0)),
        ],
        out_shape=[
            jax.ShapeDtypeStruct((np4, _PACK), jnp.float32),
            jax.ShapeDtypeStruct((1, 1), jnp.float32),
        ],
    )(x4, ubx, uby, ud, *mats)
    out_vals = jnp.concatenate([vals_sc, vals.reshape(ntc)])
    logdet = acc.reshape(()) + jnp.sum(parts_sc)
    return out_vals, logdet


# speedup vs baseline: 1.0692x; 1.0692x over previous
import functools

import jax
import jax.numpy as jnp
import numpy as np
from jax import lax
from jax.experimental import pallas as pl
from jax.experimental.pallas import tpu as pltpu
from jax.experimental.pallas import tpu_sc as plsc

_B = 32
_PACK = 4
_L = _B * _PACK
_D = _B - 1
_LD = _D * _PACK
_LOWER = -3.0
_MIN_BIN = 0.01
_MIN_DERIV = 0.01
_RED_RANGE = 6.0 - _B * _MIN_BIN
_BLOCK = 2048

_UPPER_CONST = float(
    np.float32(np.float32(_RED_RANGE) * np.float32(1.0)
               + np.float32(np.float32(_MIN_BIN) * np.float32(_B)))
    + np.float32(_LOWER))

_HI = lax.Precision.HIGHEST
_LO = lax.Precision.DEFAULT


def _seg_matrices():
    i = np.arange(_L)
    seg = i[:, None] // _B == i[None, :] // _B
    bd_lt = (seg & (i[:, None] % _B <= i[None, :] % _B)).astype(np.float32)
    bd_ones = seg.astype(np.float32)
    lts = np.concatenate([bd_lt, bd_ones], axis=1)
    e4 = (np.arange(_PACK)[:, None] == i[None, :] // _B).astype(np.float32)
    j = np.arange(_LD)
    m2 = (i[:, None] // _B == j[None, :] // _D).astype(np.float32)
    cn2 = np.concatenate([bd_ones, m2], axis=1)
    m31 = (j[:, None] // _D == i[None, :] // _B).astype(np.float32)
    c1 = (i[:, None] == _B * np.arange(_PACK)[None, :]).astype(np.float32)
    return (jnp.asarray(lts), jnp.asarray(bd_ones), jnp.asarray(e4),
            jnp.asarray(cn2), jnp.asarray(m31), jnp.asarray(c1))


def _mm(a, b, precision=_HI):
    return lax.dot_general(a, b, (((1,), (0,)), ((), ())),
                           preferred_element_type=jnp.float32,
                           precision=precision)


def _spline_body(x_ref, ubx_ref, uby_ref, ud_ref,
                 lts_ref, on_ref, e4_ref, cn2_ref, m31_ref, c1_ref,
                 vals_ref, acc_ref):
    i = pl.program_id(0)
    x4 = x_ref[:, :]
    ubx = ubx_ref[:, :]
    uby = uby_ref[:, :]
    ud = ud_ref[:, :]
    bd_ones = on_ref[:, :]
    rows = ubx.shape[0]

    il = lax.broadcasted_iota(jnp.int32, (rows, _L), 1)
    kf = jnp.bitwise_and(il, _B - 1).astype(jnp.float32)
    kterm = _MIN_BIN * (kf + 1.0) + _LOWER

    xb = _mm(x4, e4_ref[:, :])

    ex = jnp.exp(ubx)
    ey = jnp.exp(uby)
    cs2x = _mm(ex, lts_ref[:, :])
    cs2y = _mm(ey, lts_ref[:, :])
    totx = cs2x[:, _L:]
    toty = cs2y[:, _L:]
    posx = _RED_RANGE * (cs2x[:, :_L] / totx) + kterm
    cqy = _RED_RANGE * (cs2y[:, :_L] / toty) + kterm

    in_range = jnp.logical_and(xb > _LOWER, xb < _UPPER_CONST)
    xs = jnp.clip(xb, _LOWER, _UPPER_CONST)

    c = (posx < xs).astype(jnp.float32)
    cn2 = _mm(c, cn2_ref[:, :], _LO)
    cnt = cn2[:, :_L]
    cnt124 = cn2[:, _L:]

    oh_lo = (kf == cnt - 1.0).astype(jnp.float32)
    oh_hi = (kf == cnt).astype(jnp.float32)
    first = cnt == 0.0
    last = cnt == jnp.float32(_B - 1)

    lox = _mm(posx * oh_lo, bd_ones)
    ylo = _mm(cqy * oh_lo, bd_ones)
    lower_x = jnp.where(first, jnp.float32(_LOWER), lox)
    lower_y = jnp.where(first, jnp.float32(_LOWER), ylo)

    dex = _mm(ex * oh_hi, bd_ones, _LO)
    dey = _mm(ey * oh_hi, bd_ones, _LO)
    delta_x = _RED_RANGE * (dex / totx) + _MIN_BIN
    delta_y = jnp.where(last, jnp.float32(_UPPER_CONST) - lower_y,
                        _RED_RANGE * (dey / toty) + _MIN_BIN)

    il124 = lax.broadcasted_iota(jnp.int32, (rows, _LD), 1)
    j31 = (il124 - _D * (il124 // _D)).astype(jnp.float32)
    ohd_lo = (j31 == cnt124 - 1.0).astype(jnp.float32)
    ohd_hi = (j31 == cnt124).astype(jnp.float32)
    ulo = _mm(ud * ohd_lo, m31_ref[:, :], _LO)
    uhi = _mm(ud * ohd_hi, m31_ref[:, :], _LO)

    def softplus(v):
        return jnp.maximum(v, 0.0) + jnp.log(1.0 + jnp.exp(-jnp.abs(v)))

    lower_d = jnp.where(first, 1.0, softplus(ulo) + _MIN_DERIV)
    upper_d = jnp.where(last, 1.0, softplus(uhi) + _MIN_DERIV)

    r_dx = 1.0 / delta_x
    slope = delta_y * r_dx
    alpha = jnp.clip((xs - lower_x) * r_dx, 0.0, 1.0)
    a2 = alpha * alpha
    om = 1.0 - alpha
    beta = alpha * om
    gamma = om * om
    eps = upper_d + lower_d - 2.0 * slope
    r_den = 1.0 / (slope + eps * beta)
    val_s = lower_y + delta_y * (slope * a2 + lower_d * beta) * r_den
    der_s = slope * slope * (upper_d * a2 + 2.0 * slope * beta
                             + lower_d * gamma) * (r_den * r_den)

    val = jnp.where(in_range, val_s, xb)
    der = jnp.where(in_range, der_s, 1.0)

    vals_ref[:, :] = _mm(val * (kf == 0.0).astype(jnp.float32), c1_ref[:, :])

    part = jnp.sum(jnp.log(jnp.abs(der)), axis=0, keepdims=True)
    part = jnp.sum(part, axis=1, keepdims=True) * (1.0 / _B)

    @pl.when(i == 0)
    def _init():
        acc_ref[:, :] = jnp.zeros((1, 1), jnp.float32)

    acc_ref[:, :] += part


_NSC = 49152
_NW = 32
_SCL = 16
_ROWS_W = _NSC // _NW
_CHUNKS_W = _ROWS_W // _SCL

_LN2 = float(np.log(2.0))
_SQRT2 = float(np.sqrt(2.0))


def _sc_log(y):
    bits = plsc.bitcast(y, jnp.int32)
    e = jnp.right_shift(bits, 23) - 127
    m = plsc.bitcast(jnp.bitwise_or(jnp.bitwise_and(bits, 0x007FFFFF),
                                    0x3F800000), jnp.float32)
    big = m > _SQRT2
    m = jnp.where(big, m * 0.5, m)
    ef = e.astype(jnp.float32) + jnp.where(big, 1.0, 0.0)
    t = (m - 1.0) / (m + 1.0)
    t2 = t * t
    p = 1.0 + t2 * (1.0 / 3.0 + t2 * (1.0 / 5.0 + t2 * (1.0 / 7.0
                                                        + t2 / 9.0)))
    return 2.0 * t * p + _LN2 * ef


def _sc_softplus(v):
    return jnp.maximum(v, 0.0) + _sc_log(1.0 + jnp.exp(-jnp.abs(v)))


def _sc_spline(x_hbm, ubx_hbm, uby_hbm, ud_hbm, vals_hbm, part_hbm,
               xv, ubv, ubyv, udv, valv, partv):
    wid = lax.axis_index("s") * 2 + lax.axis_index("c")
    iota = lax.broadcasted_iota(jnp.int32, (_SCL,), 0)
    row_off32 = iota * _B
    row_off31 = iota * _D
    zeros = jnp.zeros((_SCL,), jnp.float32)

    def chunk_body(cki, acc):
        row0 = wid * _ROWS_W + cki * _SCL
        pltpu.sync_copy(x_hbm.at[pl.ds(row0, _SCL)], xv)
        pltpu.sync_copy(ubx_hbm.at[pl.ds(row0, _SCL)], ubv)
        pltpu.sync_copy(uby_hbm.at[pl.ds(row0, _SCL)], ubyv)
        pltpu.sync_copy(ud_hbm.at[pl.ds(row0, _SCL)], udv)

        xvec = xv[...]

        def tot_body(k, carry):
            tx, ty = carry
            ksp = iota * 0 + k
            exk = jnp.exp(plsc.load_gather(ubv, [iota, ksp]))
            eyk = jnp.exp(plsc.load_gather(ubyv, [iota, ksp]))
            return tx + exk, ty + eyk

        totx, toty = lax.fori_loop(0, _B, tot_body, (zeros, zeros))
        rtx = _RED_RANGE / totx
        rty = _RED_RANGE / toty

        xs = jnp.clip(xvec, _LOWER, _UPPER_CONST)
        in_range = jnp.logical_and(xvec > _LOWER, xvec < _UPPER_CONST)

        def scan_body(k, carry):
            (cumx, cumy, cnt, low_x, low_y, dex, dey, found) = carry
            ksp = iota * 0 + k
            exk = jnp.exp(plsc.load_gather(ubv, [iota, ksp]))
            eyk = jnp.exp(plsc.load_gather(ubyv, [iota, ksp]))
            cumx = cumx + exk
            cumy = cumy + eyk
            kshift = _MIN_BIN * (k + 1).astype(jnp.float32) + _LOWER
            pos_k = cumx * rtx + kshift
            cq_k = cumy * rty + kshift
            cmask = pos_k < xs
            cnt = cnt + jnp.where(cmask, 1.0, 0.0)
            low_x = jnp.where(cmask, pos_k, low_x)
            low_y = jnp.where(cmask, cq_k, low_y)
            newly = jnp.logical_and(jnp.logical_not(cmask), found == 0.0)
            dex = jnp.where(newly, exk * rtx + _MIN_BIN, dex)
            dey = jnp.where(newly, eyk * rty + _MIN_BIN, dey)
            found = jnp.where(newly, 1.0, found)
            return (cumx, cumy, cnt, low_x, low_y, dex, dey, found)

        init = (zeros, zeros, zeros, zeros + _LOWER, zeros + _LOWER,
                zeros + 1.0, zeros + 1.0, zeros)
        (_, _, cnt, lower_x, lower_y, delta_x, dey, _) = lax.fori_loop(
            0, _B, scan_body, init)

        cnti = cnt.astype(jnp.int32)
        first = cnt == 0.0
        last = cnt == jnp.float32(_B - 1)
        delta_y = jnp.where(last, _UPPER_CONST - lower_y, dey)

        idx_lo = jnp.maximum(cnti - 1, 0)
        idx_hi = jnp.minimum(cnti, _D - 1)
        ulo = plsc.load_gather(udv, [iota, idx_lo])
        uhi = plsc.load_gather(udv, [iota, idx_hi])
        lower_d = jnp.where(first, 1.0, _sc_softplus(ulo) + _MIN_DERIV)
        upper_d = jnp.where(last, 1.0, _sc_softplus(uhi) + _MIN_DERIV)

        r_dx = 1.0 / delta_x
        slope = delta_y * r_dx
        alpha = jnp.clip((xs - lower_x) * r_dx, 0.0, 1.0)
        a2 = alpha * alpha
        om = 1.0 - alpha
        beta = alpha * om
        gamma = om * om
        eps = upper_d + lower_d - 2.0 * slope
        r_den = 1.0 / (slope + eps * beta)
        val_s = lower_y + delta_y * (slope * a2 + lower_d * beta) * r_den
        der_s = slope * slope * (upper_d * a2 + 2.0 * slope * beta
                                 + lower_d * gamma) * (r_den * r_den)
        val = jnp.where(in_range, val_s, xvec)
        der = jnp.where(in_range, der_s, 1.0)

        valv[...] = val
        pltpu.sync_copy(valv, vals_hbm.at[pl.ds(row0, _SCL)])
        return acc + _sc_log(jnp.abs(der))

    acc = lax.fori_loop(0, _CHUNKS_W, chunk_body, zeros)
    partv[...] = acc
    pltpu.sync_copy(partv, part_hbm.at[wid])


def _sc_call(x_sc, ubx_flat, uby_flat, ud_flat):
    mesh = plsc.VectorSubcoreMesh(core_axis_name="c", subcore_axis_name="s")
    f = functools.partial(
        pl.kernel, mesh=mesh,
        compiler_params=pltpu.CompilerParams(needs_layout_passes=False),
        out_type=[jax.ShapeDtypeStruct((_NSC,), jnp.float32),
                  jax.ShapeDtypeStruct((_NW, _SCL), jnp.float32)],
        scratch_types=[
            pltpu.VMEM((_SCL,), jnp.float32),
            pltpu.VMEM((_SCL, _B), jnp.float32),
            pltpu.VMEM((_SCL, _B), jnp.float32),
            pltpu.VMEM((_SCL, _D), jnp.float32),
            pltpu.VMEM((_SCL,), jnp.float32),
            pltpu.VMEM((_SCL,), jnp.float32),
        ],
    )(_sc_spline)
    return f(x_sc, ubx_flat, uby_flat, ud_flat)


@jax.jit
def kernel(x, unconst_bin_size_x, unconst_bin_size_y, unconst_derivs):
    n = x.shape[0]

    vals_sc, parts_sc = _sc_call(
        x[:_NSC],
        unconst_bin_size_x[:_NSC],
        unconst_bin_size_y[:_NSC],
        unconst_derivs[:_NSC])

    ntc = n - _NSC
    np4 = ntc // _PACK
    r = _BLOCK
    grid = np4 // r
    x4 = x[_NSC:].reshape(np4, _PACK)
    ubx = unconst_bin_size_x[_NSC:].reshape(np4, _L)
    uby = unconst_bin_size_y[_NSC:].reshape(np4, _L)
    ud = unconst_derivs[_NSC:].reshape(np4, _LD)
    mats = _seg_matrices()

    const_spec = [
        pl.BlockSpec(m.shape, lambda i: (0, 0)) for m in mats
    ]
    vals, acc = pl.pallas_call(
        _spline_body,
        grid=(grid,),
        in_specs=[
            pl.BlockSpec((r, _PACK), lambda i: (i, 0)),
            pl.BlockSpec((r, _L), lambda i: (i, 0)),
            pl.BlockSpec((r, _L), lambda i: (i, 0)),
            pl.BlockSpec((r, _LD), lambda i: (i, 0)),
        ] + const_spec,
        out_specs=[
            pl.BlockSpec((r, _PACK), lambda i: (i, 0)),
            pl.BlockSpec((1, 1), lambda i: (0, 0)),
        ],
        out_shape=[
            jax.ShapeDtypeStruct((np4, _PACK), jnp.float32),
            jax.ShapeDtypeStruct((1, 1), jnp.float32),
        ],
    )(x4, ubx, uby, ud, *mats)
    out_vals = jnp.concatenate([vals_sc, vals.reshape(ntc)])
    logdet = acc.reshape(()) + jnp.sum(parts_sc)
    return out_vals, logdet


# speedup vs baseline: 1.1148x; 1.0426x over previous
import functools

import jax
import jax.numpy as jnp
import numpy as np
from jax import lax
from jax.experimental import pallas as pl
from jax.experimental.pallas import tpu as pltpu
from jax.experimental.pallas import tpu_sc as plsc

_B = 32
_PACK = 4
_L = _B * _PACK
_D = _B - 1
_LD = _D * _PACK
_LOWER = -3.0
_MIN_BIN = 0.01
_MIN_DERIV = 0.01
_RED_RANGE = 6.0 - _B * _MIN_BIN
_BLOCK = 2048

_UPPER_CONST = float(
    np.float32(np.float32(_RED_RANGE) * np.float32(1.0)
               + np.float32(np.float32(_MIN_BIN) * np.float32(_B)))
    + np.float32(_LOWER))

_HI = lax.Precision.HIGHEST
_LO = lax.Precision.DEFAULT


def _seg_matrices():
    i = np.arange(_L)
    seg = i[:, None] // _B == i[None, :] // _B
    bd_lt = (seg & (i[:, None] % _B <= i[None, :] % _B)).astype(np.float32)
    bd_ones = seg.astype(np.float32)
    lts = np.concatenate([bd_lt, bd_ones], axis=1)
    e4 = (np.arange(_PACK)[:, None] == i[None, :] // _B).astype(np.float32)
    j = np.arange(_LD)
    m2 = (i[:, None] // _B == j[None, :] // _D).astype(np.float32)
    cn2 = np.concatenate([bd_ones, m2], axis=1)
    m31 = (j[:, None] // _D == i[None, :] // _B).astype(np.float32)
    c1 = (i[:, None] == _B * np.arange(_PACK)[None, :]).astype(np.float32)
    return (jnp.asarray(lts), jnp.asarray(bd_ones), jnp.asarray(e4),
            jnp.asarray(cn2), jnp.asarray(m31), jnp.asarray(c1))


def _mm(a, b, precision=_HI):
    return lax.dot_general(a, b, (((1,), (0,)), ((), ())),
                           preferred_element_type=jnp.float32,
                           precision=precision)


def _spline_body(x_ref, ubx_ref, uby_ref, ud_ref,
                 lts_ref, on_ref, e4_ref, cn2_ref, m31_ref, c1_ref,
                 vals_ref, acc_ref):
    i = pl.program_id(0)
    x4 = x_ref[:, :]
    ubx = ubx_ref[:, :]
    uby = uby_ref[:, :]
    ud = ud_ref[:, :]
    bd_ones = on_ref[:, :]
    rows = ubx.shape[0]

    il = lax.broadcasted_iota(jnp.int32, (rows, _L), 1)
    kf = jnp.bitwise_and(il, _B - 1).astype(jnp.float32)
    kterm = _MIN_BIN * (kf + 1.0) + _LOWER

    xb = _mm(x4, e4_ref[:, :])

    ex = jnp.exp(ubx)
    ey = jnp.exp(uby)
    cs2x = _mm(ex, lts_ref[:, :])
    cs2y = _mm(ey, lts_ref[:, :])
    totx = cs2x[:, _L:]
    toty = cs2y[:, _L:]
    posx = _RED_RANGE * (cs2x[:, :_L] / totx) + kterm
    cqy = _RED_RANGE * (cs2y[:, :_L] / toty) + kterm

    in_range = jnp.logical_and(xb > _LOWER, xb < _UPPER_CONST)
    xs = jnp.clip(xb, _LOWER, _UPPER_CONST)

    c = (posx < xs).astype(jnp.float32)
    cn2 = _mm(c, cn2_ref[:, :], _LO)
    cnt = cn2[:, :_L]
    cnt124 = cn2[:, _L:]

    oh_lo = (kf == cnt - 1.0).astype(jnp.float32)
    oh_hi = (kf == cnt).astype(jnp.float32)
    first = cnt == 0.0
    last = cnt == jnp.float32(_B - 1)

    lox = _mm(posx * oh_lo, bd_ones)
    ylo = _mm(cqy * oh_lo, bd_ones)
    lower_x = jnp.where(first, jnp.float32(_LOWER), lox)
    lower_y = jnp.where(first, jnp.float32(_LOWER), ylo)

    dex = _mm(ex * oh_hi, bd_ones, _LO)
    dey = _mm(ey * oh_hi, bd_ones, _LO)
    delta_x = _RED_RANGE * (dex / totx) + _MIN_BIN
    delta_y = jnp.where(last, jnp.float32(_UPPER_CONST) - lower_y,
                        _RED_RANGE * (dey / toty) + _MIN_BIN)

    il124 = lax.broadcasted_iota(jnp.int32, (rows, _LD), 1)
    j31 = (il124 - _D * (il124 // _D)).astype(jnp.float32)
    ohd_lo = (j31 == cnt124 - 1.0).astype(jnp.float32)
    ohd_hi = (j31 == cnt124).astype(jnp.float32)
    ulo = _mm(ud * ohd_lo, m31_ref[:, :], _LO)
    uhi = _mm(ud * ohd_hi, m31_ref[:, :], _LO)

    def softplus(v):
        return jnp.maximum(v, 0.0) + jnp.log(1.0 + jnp.exp(-jnp.abs(v)))

    lower_d = jnp.where(first, 1.0, softplus(ulo) + _MIN_DERIV)
    upper_d = jnp.where(last, 1.0, softplus(uhi) + _MIN_DERIV)

    r_dx = 1.0 / delta_x
    slope = delta_y * r_dx
    alpha = jnp.clip((xs - lower_x) * r_dx, 0.0, 1.0)
    a2 = alpha * alpha
    om = 1.0 - alpha
    beta = alpha * om
    gamma = om * om
    eps = upper_d + lower_d - 2.0 * slope
    r_den = 1.0 / (slope + eps * beta)
    val_s = lower_y + delta_y * (slope * a2 + lower_d * beta) * r_den
    der_s = slope * slope * (upper_d * a2 + 2.0 * slope * beta
                             + lower_d * gamma) * (r_den * r_den)

    val = jnp.where(in_range, val_s, xb)
    der = jnp.where(in_range, der_s, 1.0)

    vals_ref[:, :] = _mm(val * (kf == 0.0).astype(jnp.float32), c1_ref[:, :])

    part = jnp.sum(jnp.log(jnp.abs(der)), axis=0, keepdims=True)
    part = jnp.sum(part, axis=1, keepdims=True) * (1.0 / _B)

    @pl.when(i == 0)
    def _init():
        acc_ref[:, :] = jnp.zeros((1, 1), jnp.float32)

    acc_ref[:, :] += part


_NSC = 65536
_NW = 32
_SCL = 16
_ROWS_W = _NSC // _NW
_CHUNKS_W = _ROWS_W // _SCL

_LN2 = float(np.log(2.0))
_SQRT2 = float(np.sqrt(2.0))


def _sc_log(y):
    bits = plsc.bitcast(y, jnp.int32)
    e = jnp.right_shift(bits, 23) - 127
    m = plsc.bitcast(jnp.bitwise_or(jnp.bitwise_and(bits, 0x007FFFFF),
                                    0x3F800000), jnp.float32)
    big = m > _SQRT2
    m = jnp.where(big, m * 0.5, m)
    ef = e.astype(jnp.float32) + jnp.where(big, 1.0, 0.0)
    t = (m - 1.0) / (m + 1.0)
    t2 = t * t
    p = 1.0 + t2 * (1.0 / 3.0 + t2 * (1.0 / 5.0 + t2 * (1.0 / 7.0
                                                        + t2 / 9.0)))
    return 2.0 * t * p + _LN2 * ef


def _sc_softplus(v):
    return jnp.maximum(v, 0.0) + _sc_log(1.0 + jnp.exp(-jnp.abs(v)))


def _sc_spline(x_hbm, ubx_hbm, uby_hbm, ud_hbm, vals_hbm, part_hbm,
               xv, ubv, ubyv, udv, valv, partv):
    wid = lax.axis_index("s") * 2 + lax.axis_index("c")
    iota = lax.broadcasted_iota(jnp.int32, (_SCL,), 0)
    row_off32 = iota * _B
    row_off31 = iota * _D
    zeros = jnp.zeros((_SCL,), jnp.float32)

    def chunk_body(cki, acc):
        row0 = wid * _ROWS_W + cki * _SCL
        pltpu.sync_copy(x_hbm.at[pl.ds(row0, _SCL)], xv)
        pltpu.sync_copy(ubx_hbm.at[pl.ds(row0, _SCL)], ubv)
        pltpu.sync_copy(uby_hbm.at[pl.ds(row0, _SCL)], ubyv)
        pltpu.sync_copy(ud_hbm.at[pl.ds(row0, _SCL)], udv)

        xvec = xv[...]

        def tot_body(k, carry):
            tx, ty = carry
            ksp = iota * 0 + k
            exk = jnp.exp(plsc.load_gather(ubv, [iota, ksp]))
            eyk = jnp.exp(plsc.load_gather(ubyv, [iota, ksp]))
            return tx + exk, ty + eyk

        totx, toty = lax.fori_loop(0, _B, tot_body, (zeros, zeros))
        rtx = _RED_RANGE / totx
        rty = _RED_RANGE / toty

        xs = jnp.clip(xvec, _LOWER, _UPPER_CONST)
        in_range = jnp.logical_and(xvec > _LOWER, xvec < _UPPER_CONST)

        def scan_body(k, carry):
            (cumx, cumy, cnt, low_x, low_y, dex, dey, found) = carry
            ksp = iota * 0 + k
            exk = jnp.exp(plsc.load_gather(ubv, [iota, ksp]))
            eyk = jnp.exp(plsc.load_gather(ubyv, [iota, ksp]))
            cumx = cumx + exk
            cumy = cumy + eyk
            kshift = _MIN_BIN * (k + 1).astype(jnp.float32) + _LOWER
            pos_k = cumx * rtx + kshift
            cq_k = cumy * rty + kshift
            cmask = pos_k < xs
            cnt = cnt + jnp.where(cmask, 1.0, 0.0)
            low_x = jnp.where(cmask, pos_k, low_x)
            low_y = jnp.where(cmask, cq_k, low_y)
            newly = jnp.logical_and(jnp.logical_not(cmask), found == 0.0)
            dex = jnp.where(newly, exk * rtx + _MIN_BIN, dex)
            dey = jnp.where(newly, eyk * rty + _MIN_BIN, dey)
            found = jnp.where(newly, 1.0, found)
            return (cumx, cumy, cnt, low_x, low_y, dex, dey, found)

        init = (zeros, zeros, zeros, zeros + _LOWER, zeros + _LOWER,
                zeros + 1.0, zeros + 1.0, zeros)
        (_, _, cnt, lower_x, lower_y, delta_x, dey, _) = lax.fori_loop(
            0, _B, scan_body, init)

        cnti = cnt.astype(jnp.int32)
        first = cnt == 0.0
        last = cnt == jnp.float32(_B - 1)
        delta_y = jnp.where(last, _UPPER_CONST - lower_y, dey)

        idx_lo = jnp.maximum(cnti - 1, 0)
        idx_hi = jnp.minimum(cnti, _D - 1)
        ulo = plsc.load_gather(udv, [iota, idx_lo])
        uhi = plsc.load_gather(udv, [iota, idx_hi])
        lower_d = jnp.where(first, 1.0, _sc_softplus(ulo) + _MIN_DERIV)
        upper_d = jnp.where(last, 1.0, _sc_softplus(uhi) + _MIN_DERIV)

        r_dx = 1.0 / delta_x
        slope = delta_y * r_dx
        alpha = jnp.clip((xs - lower_x) * r_dx, 0.0, 1.0)
        a2 = alpha * alpha
        om = 1.0 - alpha
        beta = alpha * om
        gamma = om * om
        eps = upper_d + lower_d - 2.0 * slope
        r_den = 1.0 / (slope + eps * beta)
        val_s = lower_y + delta_y * (slope * a2 + lower_d * beta) * r_den
        der_s = slope * slope * (upper_d * a2 + 2.0 * slope * beta
                                 + lower_d * gamma) * (r_den * r_den)
        val = jnp.where(in_range, val_s, xvec)
        der = jnp.where(in_range, der_s, 1.0)

        valv[...] = val
        pltpu.sync_copy(valv, vals_hbm.at[pl.ds(row0, _SCL)])
        return acc + _sc_log(jnp.abs(der))

    acc = lax.fori_loop(0, _CHUNKS_W, chunk_body, zeros)
    partv[...] = acc
    pltpu.sync_copy(partv, part_hbm.at[wid])


def _sc_call(x_sc, ubx_flat, uby_flat, ud_flat):
    mesh = plsc.VectorSubcoreMesh(core_axis_name="c", subcore_axis_name="s")
    f = functools.partial(
        pl.kernel, mesh=mesh,
        compiler_params=pltpu.CompilerParams(needs_layout_passes=False),
        out_type=[jax.ShapeDtypeStruct((_NSC,), jnp.float32),
                  jax.ShapeDtypeStruct((_NW, _SCL), jnp.float32)],
        scratch_types=[
            pltpu.VMEM((_SCL,), jnp.float32),
            pltpu.VMEM((_SCL, _B), jnp.float32),
            pltpu.VMEM((_SCL, _B), jnp.float32),
            pltpu.VMEM((_SCL, _D), jnp.float32),
            pltpu.VMEM((_SCL,), jnp.float32),
            pltpu.VMEM((_SCL,), jnp.float32),
        ],
    )(_sc_spline)
    return f(x_sc, ubx_flat, uby_flat, ud_flat)


@jax.jit
def kernel(x, unconst_bin_size_x, unconst_bin_size_y, unconst_derivs):
    n = x.shape[0]

    vals_sc, parts_sc = _sc_call(
        x[:_NSC],
        unconst_bin_size_x[:_NSC],
        unconst_bin_size_y[:_NSC],
        unconst_derivs[:_NSC])

    ntc = n - _NSC
    np4 = ntc // _PACK
    r = _BLOCK
    grid = np4 // r
    x4 = x[_NSC:].reshape(np4, _PACK)
    ubx = unconst_bin_size_x[_NSC:].reshape(np4, _L)
    uby = unconst_bin_size_y[_NSC:].reshape(np4, _L)
    ud = unconst_derivs[_NSC:].reshape(np4, _LD)
    mats = _seg_matrices()

    const_spec = [
        pl.BlockSpec(m.shape, lambda i: (0, 0)) for m in mats
    ]
    vals, acc = pl.pallas_call(
        _spline_body,
        grid=(grid,),
        in_specs=[
            pl.BlockSpec((r, _PACK), lambda i: (i, 0)),
            pl.BlockSpec((r, _L), lambda i: (i, 0)),
            pl.BlockSpec((r, _L), lambda i: (i, 0)),
            pl.BlockSpec((r, _LD), lambda i: (i, 0)),
        ] + const_spec,
        out_specs=[
            pl.BlockSpec((r, _PACK), lambda i: (i, 0)),
            pl.BlockSpec((1, 1), lambda i: (0, 0)),
        ],
        out_shape=[
            jax.ShapeDtypeStruct((np4, _PACK), jnp.float32),
            jax.ShapeDtypeStruct((1, 1), jnp.float32),
        ],
    )(x4, ubx, uby, ud, *mats)
    out_vals = jnp.concatenate([vals_sc, vals.reshape(ntc)])
    logdet = acc.reshape(()) + jnp.sum(parts_sc)
    return out_vals, logdet
